# Initial kernel scaffold; baseline (speedup 1.0000x reference)
import jax
import jax.numpy as jnp
from jax.experimental import pallas as pl


def kernel(x, edge_index, W1, b1, Wr, br, Wx, bx):
    raise NotImplementedError("write your pallas kernel here")


# speedup vs baseline: 9.8177x; 9.8177x over previous
import functools

import jax
import jax.numpy as jnp
from jax import lax
from jax.experimental import pallas as pl
from jax.experimental.pallas import tpu as pltpu
from jax.experimental.pallas import tpu_sc as plsc

N = 10000
E = 320000
D = 128
DEPTH = 3

NC = 2
NS = 16
NW = NC * NS
EPW = E // NW
CH = 128
NCHK = -(-EPW // CH)
EPW_PAD = NCHK * CH
NOUT = 10240
RPT = NOUT // NS

_mesh = plsc.VectorSubcoreMesh(
    core_axis_name="c", subcore_axis_name="s", num_cores=NC, num_subcores=NS)


def _zero_rows_buf(rows, nrow, width):
    zeros16 = jnp.zeros((16,), jnp.float32)

    def body(i, carry):
        for j in range(width // 16):
            rows[i, pl.ds(j * 16, 16)] = zeros16
        return carry

    lax.fori_loop(0, nrow, body, 0)


@functools.partial(
    pl.kernel,
    out_type=jax.ShapeDtypeStruct((NC, NOUT, 16), jnp.float32),
    mesh=_mesh,
    scratch_types=[
        pltpu.VMEM((NCHK, CH), jnp.int32),
        pltpu.VMEM((CH, 16), jnp.float32),
        pltpu.VMEM_SHARED((NOUT, 16), jnp.float32),
    ],
)
def _degree_kernel(dst_ref, out_ref, idx_d, rows, acc):
    c = lax.axis_index("c")
    s = lax.axis_index("s")
    wid = c * NS + s
    pltpu.sync_copy(dst_ref.at[wid], idx_d)
    _zero_rows_buf(rows, CH, 16)
    r0 = s * RPT
    for t in range(RPT // CH):
        pltpu.sync_copy(rows, acc.at[pl.ds(r0 + t * CH, CH)])
    plsc.subcore_barrier()
    ones16 = jnp.ones((16,), jnp.float32)

    def fill(i, carry):
        rows[i] = ones16
        return carry

    lax.fori_loop(0, CH, fill, 0)

    def chunk(j, carry):
        pltpu.sync_copy(rows, acc.at[idx_d.at[j]], add=True)
        return carry

    lax.fori_loop(0, NCHK, chunk, 0)
    plsc.subcore_barrier()
    pltpu.sync_copy(acc.at[pl.ds(r0, RPT)], out_ref.at[c, pl.ds(r0, RPT)])


@functools.partial(
    pl.kernel,
    out_type=jax.ShapeDtypeStruct((NC, NOUT, D), jnp.float32),
    mesh=_mesh,
    scratch_types=[
        pltpu.VMEM((NCHK, CH), jnp.int32),
        pltpu.VMEM((NCHK, CH), jnp.int32),
        pltpu.VMEM((CH, D), jnp.float32),
        pltpu.VMEM_SHARED((NOUT, D), jnp.float32),
        pltpu.SemaphoreType.DMA,
    ],
)
def _propagate_kernel(src_ref, dst_ref, y_ref, out_ref, idx_s, idx_d, rows,
                      acc, sem):
    c = lax.axis_index("c")
    s = lax.axis_index("s")
    wid = c * NS + s
    pltpu.sync_copy(src_ref.at[wid], idx_s)
    pltpu.sync_copy(dst_ref.at[wid], idx_d)
    _zero_rows_buf(rows, CH, D)
    r0 = s * RPT
    for t in range(RPT // CH):
        pltpu.sync_copy(rows, acc.at[pl.ds(r0 + t * CH, CH)])
    plsc.subcore_barrier()

    def chunk(j, carry):
        pltpu.async_copy(y_ref.at[idx_s.at[j]], rows, sem).wait()
        pltpu.sync_copy(rows, acc.at[idx_d.at[j]], add=True)
        return carry

    lax.fori_loop(0, NCHK, chunk, 0)
    plsc.subcore_barrier()
    pltpu.sync_copy(acc.at[pl.ds(r0, RPT)], out_ref.at[c, pl.ds(r0, RPT)])


BM = 1024


def _tc_first_body(d0_ref, d1_ref, x_ref, w_ref, dis_ref, y_ref):
    dis = lax.rsqrt(d0_ref[...] + d1_ref[...] + 1.0)
    dis_ref[...] = dis
    y_ref[...] = dis * jnp.dot(x_ref[...], w_ref[...],
                               preferred_element_type=jnp.float32)


_tc_first = pl.pallas_call(
    _tc_first_body,
    grid=(NOUT // BM,),
    in_specs=[
        pl.BlockSpec((BM, 1), lambda i: (i, 0)),
        pl.BlockSpec((BM, 1), lambda i: (i, 0)),
        pl.BlockSpec((BM, D), lambda i: (i, 0)),
        pl.BlockSpec((D, D), lambda i: (0, 0)),
    ],
    out_specs=[
        pl.BlockSpec((BM, 1), lambda i: (i, 0)),
        pl.BlockSpec((BM, D), lambda i: (i, 0)),
    ],
    out_shape=[
        jax.ShapeDtypeStruct((NOUT, 1), jnp.float32),
        jax.ShapeDtypeStruct((NOUT, D), jnp.float32),
    ],
)


def _tc_mid_body(residual, dis_ref, z0_ref, z1_ref, y_ref, b_ref, h_ref,
                 w_ref, hout_ref, yout_ref):
    dis = dis_ref[...]
    t = dis * (z0_ref[...] + z1_ref[...] + y_ref[...]) + b_ref[...]
    h = jnp.maximum(t, 0.0)
    if residual:
        h = h + h_ref[...]
    hout_ref[...] = h
    yout_ref[...] = dis * jnp.dot(h, w_ref[...],
                                  preferred_element_type=jnp.float32)


def _make_tc_mid(residual):
    return pl.pallas_call(
        functools.partial(_tc_mid_body, residual),
        grid=(NOUT // BM,),
        in_specs=[
            pl.BlockSpec((BM, 1), lambda i: (i, 0)),
            pl.BlockSpec((BM, D), lambda i: (i, 0)),
            pl.BlockSpec((BM, D), lambda i: (i, 0)),
            pl.BlockSpec((BM, D), lambda i: (i, 0)),
            pl.BlockSpec((1, D), lambda i: (0, 0)),
            pl.BlockSpec((BM, D), lambda i: (i, 0)),
            pl.BlockSpec((D, D), lambda i: (0, 0)),
        ],
        out_specs=[
            pl.BlockSpec((BM, D), lambda i: (i, 0)),
            pl.BlockSpec((BM, D), lambda i: (i, 0)),
        ],
        out_shape=[
            jax.ShapeDtypeStruct((NOUT, D), jnp.float32),
            jax.ShapeDtypeStruct((NOUT, D), jnp.float32),
        ],
    )


_tc_mid_nores = _make_tc_mid(False)
_tc_mid_res = _make_tc_mid(True)


def _tc_last_body(dis_ref, z0_ref, z1_ref, y_ref, b_ref, out_ref):
    out_ref[...] = (dis_ref[...] * (z0_ref[...] + z1_ref[...] + y_ref[...])
                    + b_ref[...])


_tc_last = pl.pallas_call(
    _tc_last_body,
    grid=(NOUT // BM,),
    in_specs=[
        pl.BlockSpec((BM, 1), lambda i: (i, 0)),
        pl.BlockSpec((BM, D), lambda i: (i, 0)),
        pl.BlockSpec((BM, D), lambda i: (i, 0)),
        pl.BlockSpec((BM, D), lambda i: (i, 0)),
        pl.BlockSpec((1, D), lambda i: (0, 0)),
    ],
    out_specs=pl.BlockSpec((BM, D), lambda i: (i, 0)),
    out_shape=jax.ShapeDtypeStruct((NOUT, D), jnp.float32),
)


def kernel(x, edge_index, W1, b1, Wr, br, Wx, bx):
    src = edge_index[0].astype(jnp.int32).reshape(NW, EPW)
    dst = edge_index[1].astype(jnp.int32).reshape(NW, EPW)
    pad = EPW_PAD - EPW
    src = jnp.pad(src, ((0, 0), (0, pad))).reshape(NW, NCHK, CH)
    dst = jnp.pad(dst, ((0, 0), (0, pad)),
                  constant_values=N).reshape(NW, NCHK, CH)
    xp = jnp.pad(x, ((0, NOUT - N), (0, 0)))

    degp = _degree_kernel(dst)
    d0 = degp[0, :, 0:1]
    d1 = degp[1, :, 0:1]

    b1r = b1.reshape(1, D)
    brr = br.reshape(1, D)
    bxr = bx.reshape(1, D)

    dis, y = _tc_first(d0, d1, xp, W1)

    z = _propagate_kernel(src, dst, y)
    h, y = _tc_mid_nores(dis, z[0], z[1], y, b1r, y, Wr)

    for k in range(DEPTH):
        z = _propagate_kernel(src, dst, y)
        w_next = Wr if k < DEPTH - 1 else Wx
        h, y = _tc_mid_res(dis, z[0], z[1], y, brr, h, w_next)

    z = _propagate_kernel(src, dst, y)
    out = _tc_last(dis, z[0], z[1], y, bxr)
    return out[:N]


# speedup vs baseline: 11.5333x; 1.1747x over previous
import functools

import jax
import jax.numpy as jnp
from jax import lax
from jax.experimental import pallas as pl
from jax.experimental.pallas import tpu as pltpu
from jax.experimental.pallas import tpu_sc as plsc

N = 10000
E = 320000
D = 128
DH = D // 2
DEPTH = 3

NC = 2
NS = 16
NW = NC * NS
CH = 128
NBUF = 4

EPW = E // NW
NCHK_DEG = 80
EPW_PAD = NCHK_DEG * CH

EPT = E // NS
NCHK = 160
EPT_PAD = NCHK * CH

NOUT = 10240
RPT = NOUT // NS

_mesh = plsc.VectorSubcoreMesh(
    core_axis_name="c", subcore_axis_name="s", num_cores=NC, num_subcores=NS)


def _zero_rows_buf(rows, nrow, width):
    zeros16 = jnp.zeros((16,), jnp.float32)

    def body(i, carry):
        for j in range(width // 16):
            rows[i, pl.ds(j * 16, 16)] = zeros16
        return carry

    lax.fori_loop(0, nrow, body, 0)


@functools.partial(
    pl.kernel,
    out_type=jax.ShapeDtypeStruct((NC, NOUT, 16), jnp.float32),
    mesh=_mesh,
    scratch_types=[
        pltpu.VMEM((NCHK_DEG, CH), jnp.int32),
        pltpu.VMEM((CH, 16), jnp.float32),
        pltpu.VMEM_SHARED((NOUT, 16), jnp.float32),
    ],
)
def _degree_kernel(dst_ref, out_ref, idx_d, rows, acc):
    c = lax.axis_index("c")
    s = lax.axis_index("s")
    wid = c * NS + s
    pltpu.sync_copy(dst_ref.at[wid], idx_d)
    _zero_rows_buf(rows, CH, 16)
    r0 = s * RPT
    for t in range(RPT // CH):
        pltpu.sync_copy(rows, acc.at[pl.ds(r0 + t * CH, CH)])
    plsc.subcore_barrier()
    ones16 = jnp.ones((16,), jnp.float32)

    def fill(i, carry):
        rows[i] = ones16
        return carry

    lax.fori_loop(0, CH, fill, 0)

    def chunk(j, carry):
        pltpu.sync_copy(rows, acc.at[idx_d.at[j]], add=True)
        return carry

    lax.fori_loop(0, NCHK_DEG, chunk, 0)
    plsc.subcore_barrier()
    pltpu.sync_copy(acc.at[pl.ds(r0, RPT)], out_ref.at[c, pl.ds(r0, RPT)])


@functools.partial(
    pl.kernel,
    out_type=jax.ShapeDtypeStruct((NC, NOUT, DH), jnp.float32),
    mesh=_mesh,
    scratch_types=[
        pltpu.VMEM((NCHK, CH), jnp.int32),
        pltpu.VMEM((NCHK, CH), jnp.int32),
        pltpu.VMEM((CH, DH), jnp.float32),
        pltpu.VMEM((CH, DH), jnp.float32),
        pltpu.VMEM((CH, DH), jnp.float32),
        pltpu.VMEM((CH, DH), jnp.float32),
        pltpu.VMEM_SHARED((NOUT, DH), jnp.float32),
        pltpu.SemaphoreType.DMA,
        pltpu.SemaphoreType.DMA,
        pltpu.SemaphoreType.DMA,
        pltpu.SemaphoreType.DMA,
    ],
    compiler_params=pltpu.CompilerParams(use_tc_tiling_on_sc=False),
)
def _propagate_kernel(src_ref, dst_ref, y_ref, out_ref, idx_s, idx_d, rows0,
                      rows1, rows2, rows3, acc, sem0, sem1, sem2, sem3):
    rows = (rows0, rows1, rows2, rows3)
    sems = (sem0, sem1, sem2, sem3)
    c = lax.axis_index("c")
    s = lax.axis_index("s")
    yc = y_ref.at[c]
    pltpu.sync_copy(src_ref.at[s], idx_s)
    pltpu.sync_copy(dst_ref.at[s], idx_d)
    _zero_rows_buf(rows0, CH, DH)
    r0 = s * RPT
    for t in range(RPT // CH):
        pltpu.sync_copy(rows0, acc.at[pl.ds(r0 + t * CH, CH)])
    plsc.subcore_barrier()

    for b in range(NBUF):
        pltpu.async_copy(yc.at[idx_s.at[b]], rows[b], sems[b])

    def chunk(jj, carry):
        for b in range(NBUF):
            j = jj * NBUF + b
            pltpu.make_async_copy(yc.at[idx_s.at[j]], rows[b],
                                  sems[b]).wait()
            pltpu.sync_copy(rows[b], acc.at[idx_d.at[j]], add=True)

            @pl.when(j + NBUF < NCHK)
            def _():
                pltpu.async_copy(yc.at[idx_s.at[j + NBUF]], rows[b],
                                 sems[b])
        return carry

    lax.fori_loop(0, NCHK // NBUF, chunk, 0)
    plsc.subcore_barrier()
    pltpu.sync_copy(acc.at[pl.ds(r0, RPT)], out_ref.at[c, pl.ds(r0, RPT)])


BM = 1024

_split_spec = pl.BlockSpec((NC, BM, DH), lambda i: (0, i, 0))
_dense_spec = pl.BlockSpec((BM, D), lambda i: (i, 0))
_dis_spec = pl.BlockSpec((BM, 1), lambda i: (i, 0))
_w_spec = pl.BlockSpec((D, D), lambda i: (0, 0))
_b_spec = pl.BlockSpec((NC, 1, DH), lambda i: (0, 0, 0))

_split_shape = jax.ShapeDtypeStruct((NC, NOUT, DH), jnp.float32)
_dense_shape = jax.ShapeDtypeStruct((NOUT, D), jnp.float32)


def _store_split(ref, v):
    ref[0] = v[:, :DH]
    ref[1] = v[:, DH:]


def _tc_first_body(d0_ref, d1_ref, x_ref, w_ref, dis_ref, y_ref):
    dis = lax.rsqrt(d0_ref[...] + d1_ref[...] + 1.0)
    dis_ref[...] = dis
    y = dis * jnp.dot(x_ref[...], w_ref[...],
                      preferred_element_type=jnp.float32)
    _store_split(y_ref, y)


_tc_first = pl.pallas_call(
    _tc_first_body,
    grid=(NOUT // BM,),
    in_specs=[_dis_spec, _dis_spec, _dense_spec, _w_spec],
    out_specs=[_dis_spec, _split_spec],
    out_shape=[jax.ShapeDtypeStruct((NOUT, 1), jnp.float32), _split_shape],
)


def _tc_mid_body(residual, dis_ref, z_ref, y_ref, b_ref, h_ref, w_ref,
                 hout_ref, yout_ref):
    dis = dis_ref[...]
    halves = []
    for c in range(NC):
        t = dis * (z_ref[c] + y_ref[c]) + b_ref[c]
        hc = jnp.maximum(t, 0.0)
        if residual:
            hc = hc + h_ref[c]
        halves.append(hc)
        hout_ref[c] = hc
    u = (jnp.dot(halves[0], w_ref[0:DH, :],
                 preferred_element_type=jnp.float32) +
         jnp.dot(halves[1], w_ref[DH:D, :],
                 preferred_element_type=jnp.float32))
    _store_split(yout_ref, dis * u)


def _make_tc_mid(residual):
    return pl.pallas_call(
        functools.partial(_tc_mid_body, residual),
        grid=(NOUT // BM,),
        in_specs=[_dis_spec, _split_spec, _split_spec, _b_spec, _split_spec,
                  _w_spec],
        out_specs=[_split_spec, _split_spec],
        out_shape=[_split_shape, _split_shape],
    )


_tc_mid_nores = _make_tc_mid(False)
_tc_mid_res = _make_tc_mid(True)


def _tc_last_body(dis_ref, z_ref, y_ref, b_ref, out_ref):
    dis = dis_ref[...]
    for c in range(NC):
        out_ref[:, c * DH:(c + 1) * DH] = (dis * (z_ref[c] + y_ref[c])
                                           + b_ref[c])


_tc_last = pl.pallas_call(
    _tc_last_body,
    grid=(NOUT // BM,),
    in_specs=[_dis_spec, _split_spec, _split_spec, _b_spec],
    out_specs=_dense_spec,
    out_shape=_dense_shape,
)


def _pad_edges(v, n_split, n_chunk, fill):
    per = E // n_split
    v = v.reshape(n_split, per)
    return jnp.pad(v, ((0, 0), (0, n_chunk * CH - per)),
                   constant_values=fill).reshape(n_split, n_chunk, CH)


def kernel(x, edge_index, W1, b1, Wr, br, Wx, bx):
    src = edge_index[0].astype(jnp.int32)
    dst = edge_index[1].astype(jnp.int32)
    dst_deg = _pad_edges(dst, NW, NCHK_DEG, N)
    src_p = _pad_edges(src, NS, NCHK, 0)
    dst_p = _pad_edges(dst, NS, NCHK, N)
    xp = jnp.pad(x, ((0, NOUT - N), (0, 0)))

    degp = _degree_kernel(dst_deg)
    d0 = degp[0, :, 0:1]
    d1 = degp[1, :, 0:1]

    b1s = b1.reshape(NC, 1, DH)
    brs = br.reshape(NC, 1, DH)
    bxs = bx.reshape(NC, 1, DH)

    dis, y = _tc_first(d0, d1, xp, W1)

    z = _propagate_kernel(src_p, dst_p, y)
    h, y = _tc_mid_nores(dis, z, y, b1s, y, Wr)

    for k in range(DEPTH):
        z = _propagate_kernel(src_p, dst_p, y)
        w_next = Wr if k < DEPTH - 1 else Wx
        h, y = _tc_mid_res(dis, z, y, brs, h, w_next)

    z = _propagate_kernel(src_p, dst_p, y)
    out = _tc_last(dis, z, y, bxs)
    return out[:N]
